# Initial kernel scaffold; baseline (speedup 1.0000x reference)
import jax
import jax.numpy as jnp
from jax.experimental import pallas as pl


def kernel(x, spiral_idx, W_point, W_r1a, b_r1a, W_r1b, b_r1b, W_mid, W_s0a, b_s0a, W_s0b, b_s0b, W_s1a, b_s1a, W_s1b, b_s1b, W_s2a, b_s2a, W_s2b, b_s2b, W_o0, b_o0, W_o1, b_o1, W_o2, b_o2):
    raise NotImplementedError("write your pallas kernel here")


# speedup vs baseline: 4.9392x; 4.9392x over previous
import functools

import jax
import jax.numpy as jnp
from jax import lax
from jax.experimental import pallas as pl
from jax.experimental.pallas import tpu as pltpu
from jax.experimental.pallas import tpu_sc as plsc

_B = 2
_V = 5000
_L = 9
_VP = 5120
_N = _B * _VP
_BR = 512

_NC, _NS = 2, 16
_NW = _NC * _NS
_VPW = _N // _NW


@functools.lru_cache(maxsize=None)
def _make_gather_sum(C, CH):
    NCH = _VPW // CH
    nc16 = C // 16
    mesh = plsc.VectorSubcoreMesh(core_axis_name="c", subcore_axis_name="s")

    @functools.partial(
        pl.kernel,
        mesh=mesh,
        out_type=jax.ShapeDtypeStruct((_N, C), jnp.float32),
        scratch_types=[
            pltpu.VMEM((_L * _VPW,), jnp.int32),
            pltpu.VMEM((_L, CH, C), jnp.float32),
            pltpu.VMEM((CH, C), jnp.float32),
            pltpu.SemaphoreType.DMA,
        ],
    )
    def gs(table_hbm, gidx_hbm, out_hbm, idx_v, buf_v, acc_v, sem):
        wid = lax.axis_index("s") * _NC + lax.axis_index("c")
        pltpu.sync_copy(gidx_hbm.at[wid], idx_v)
        for c in range(NCH):
            cps = [
                pltpu.async_copy(
                    table_hbm.at[idx_v.at[pl.ds(j * _VPW + c * CH, CH)]],
                    buf_v.at[j],
                    sem,
                )
                for j in range(_L)
            ]
            for cp in cps:
                cp.wait()

            def rbody(r, _):
                def cbody(cb, _2):
                    co = cb * 16
                    s = buf_v[0, r, pl.ds(co, 16)]
                    for j in range(1, _L):
                        s = s + buf_v[j, r, pl.ds(co, 16)]
                    acc_v[r, pl.ds(co, 16)] = s
                    return 0

                return lax.fori_loop(0, nc16, cbody, 0)

            lax.fori_loop(0, CH, rbody, 0)
            pltpu.sync_copy(acc_v, out_hbm.at[pl.ds(wid * _VPW + c * CH, CH)])

    return gs


def _k1_body(x_ref, wp_ref, w1_ref, m_ref, pfs_ref, z_ref):
    h = jnp.dot(x_ref[...], wp_ref[...], preferred_element_type=jnp.float32)
    h = jnp.maximum(h, 0.0) * m_ref[...]
    pfs_ref[...] = h
    z_ref[...] = jnp.dot(h, w1_ref[...], preferred_element_type=jnp.float32)


def _k1(xp, wp, w1, mask):
    K = xp.shape[1]
    CO = w1.shape[1]
    return pl.pallas_call(
        _k1_body,
        grid=(_N // _BR,),
        in_specs=[
            pl.BlockSpec((_BR, K), lambda i: (i, 0)),
            pl.BlockSpec((K, 256), lambda i: (0, 0)),
            pl.BlockSpec((256, CO), lambda i: (0, 0)),
            pl.BlockSpec((_BR, 1), lambda i: (i, 0)),
        ],
        out_specs=[
            pl.BlockSpec((_BR, 256), lambda i: (i, 0)),
            pl.BlockSpec((_BR, CO), lambda i: (i, 0)),
        ],
        out_shape=[
            jax.ShapeDtypeStruct((_N, 256), jnp.float32),
            jax.ShapeDtypeStruct((_N, CO), jnp.float32),
        ],
    )(xp, wp, w1, mask)


def _act_mm_body(g_ref, b_ref, m_ref, w_ref, z_ref):
    h = jnp.maximum(g_ref[...] + b_ref[...], 0.0) * m_ref[...]
    z_ref[...] = jnp.dot(h, w_ref[...], preferred_element_type=jnp.float32)


def _act_mm(g, b, mask, w):
    C = g.shape[1]
    CO = w.shape[1]
    return pl.pallas_call(
        _act_mm_body,
        grid=(_N // _BR,),
        in_specs=[
            pl.BlockSpec((_BR, C), lambda i: (i, 0)),
            pl.BlockSpec((1, C), lambda i: (0, 0)),
            pl.BlockSpec((_BR, 1), lambda i: (i, 0)),
            pl.BlockSpec((C, CO), lambda i: (0, 0)),
        ],
        out_specs=pl.BlockSpec((_BR, CO), lambda i: (i, 0)),
        out_shape=jax.ShapeDtypeStruct((_N, CO), jnp.float32),
    )(g, b, mask, w)


def _res_mm_body(g_ref, b_ref, r_ref, m_ref, w_ref, fs_ref, z_ref):
    h = jnp.maximum(g_ref[...] + b_ref[...] + r_ref[...], 0.0) * m_ref[...]
    fs_ref[...] = h
    z_ref[...] = jnp.dot(h, w_ref[...], preferred_element_type=jnp.float32)


def _res_mm(g, b, res, mask, w):
    C = g.shape[1]
    CO = w.shape[1]
    return pl.pallas_call(
        _res_mm_body,
        grid=(_N // _BR,),
        in_specs=[
            pl.BlockSpec((_BR, C), lambda i: (i, 0)),
            pl.BlockSpec((1, C), lambda i: (0, 0)),
            pl.BlockSpec((_BR, C), lambda i: (i, 0)),
            pl.BlockSpec((_BR, 1), lambda i: (i, 0)),
            pl.BlockSpec((C, CO), lambda i: (0, 0)),
        ],
        out_specs=[
            pl.BlockSpec((_BR, C), lambda i: (i, 0)),
            pl.BlockSpec((_BR, CO), lambda i: (i, 0)),
        ],
        out_shape=[
            jax.ShapeDtypeStruct((_N, C), jnp.float32),
            jax.ShapeDtypeStruct((_N, CO), jnp.float32),
        ],
    )(g, b, res, mask, w)


def _mid_body(g_ref, b_ref, r_ref, m_ref, wmid_ref, wn_ref, fs_ref, z_ref):
    h = jnp.maximum(g_ref[...] + b_ref[...] + r_ref[...], 0.0) * m_ref[...]
    fs = jnp.dot(h, wmid_ref[...], preferred_element_type=jnp.float32)
    fs_ref[...] = fs
    z_ref[...] = jnp.dot(fs, wn_ref[...], preferred_element_type=jnp.float32)


def _mid(g, b, res, mask, wmid, wn):
    CO = wn.shape[1]
    return pl.pallas_call(
        _mid_body,
        grid=(_N // _BR,),
        in_specs=[
            pl.BlockSpec((_BR, 256), lambda i: (i, 0)),
            pl.BlockSpec((1, 256), lambda i: (0, 0)),
            pl.BlockSpec((_BR, 256), lambda i: (i, 0)),
            pl.BlockSpec((_BR, 1), lambda i: (i, 0)),
            pl.BlockSpec((256, 128), lambda i: (0, 0)),
            pl.BlockSpec((128, CO), lambda i: (0, 0)),
        ],
        out_specs=[
            pl.BlockSpec((_BR, 128), lambda i: (i, 0)),
            pl.BlockSpec((_BR, CO), lambda i: (i, 0)),
        ],
        out_shape=[
            jax.ShapeDtypeStruct((_N, 128), jnp.float32),
            jax.ShapeDtypeStruct((_N, CO), jnp.float32),
        ],
    )(g, b, res, mask, wmid, wn)


def _max_body(fs_ref, m_ref, o_ref):
    m = jnp.where(m_ref[...] > 0.0, fs_ref[...], -1e30)
    o_ref[...] = jnp.max(m.reshape(_B, _VP, 128), axis=1)


def _gmax(fs, mask):
    return pl.pallas_call(
        _max_body,
        in_specs=[
            pl.BlockSpec((_N, 128), lambda: (0, 0)),
            pl.BlockSpec((_N, 1), lambda: (0, 0)),
        ],
        out_specs=pl.BlockSpec((_B, 128), lambda: (0, 0)),
        out_shape=jax.ShapeDtypeStruct((_B, 128), jnp.float32),
    )(fs, mask)


def _out_body(g_ref, b_ref, r_ref, m_ref, pfs_ref, gfs_ref,
              wo0_ref, bo0_ref, wo1_ref, bo1_ref, wo2_ref, bo2_ref, o_ref):
    fs3 = jnp.maximum(g_ref[...] + b_ref[...] + r_ref[...], 0.0) * m_ref[...]
    bsel = lax.broadcasted_iota(jnp.int32, (_B, 128), 0) == pl.program_id(0)
    grow = jnp.sum(jnp.where(bsel, gfs_ref[...], 0.0), axis=0, keepdims=True)
    t = jnp.dot(pfs_ref[...], wo0_ref[0:256], preferred_element_type=jnp.float32)
    t = t + jnp.dot(fs3, wo0_ref[256:384], preferred_element_type=jnp.float32)
    t = t + jnp.dot(grow, wo0_ref[384:512], preferred_element_type=jnp.float32)
    o = jnp.maximum(t + bo0_ref[...], 0.0)
    o = jnp.maximum(
        jnp.dot(o, wo1_ref[...], preferred_element_type=jnp.float32) + bo1_ref[...], 0.0
    )
    o_ref[...] = jnp.dot(o, wo2_ref[...], preferred_element_type=jnp.float32) + bo2_ref[...]


def _out_mlp(g, b, res, mask, pfs, gfs, wo0, bo0, wo1, bo1, wo2, bo2):
    nrb = _VP // _BR
    row = lambda b_, i: (b_ * nrb + i, 0)
    return pl.pallas_call(
        _out_body,
        grid=(_B, nrb),
        in_specs=[
            pl.BlockSpec((_BR, 128), row),
            pl.BlockSpec((1, 128), lambda b_, i: (0, 0)),
            pl.BlockSpec((_BR, 128), row),
            pl.BlockSpec((_BR, 1), row),
            pl.BlockSpec((_BR, 256), row),
            pl.BlockSpec((_B, 128), lambda b_, i: (0, 0)),
            pl.BlockSpec((512, 256), lambda b_, i: (0, 0)),
            pl.BlockSpec((1, 256), lambda b_, i: (0, 0)),
            pl.BlockSpec((256, 128), lambda b_, i: (0, 0)),
            pl.BlockSpec((1, 128), lambda b_, i: (0, 0)),
            pl.BlockSpec((128, 3), lambda b_, i: (0, 0)),
            pl.BlockSpec((1, 3), lambda b_, i: (0, 0)),
        ],
        out_specs=pl.BlockSpec((_BR, 3), row),
        out_shape=jax.ShapeDtypeStruct((_N, 3), jnp.float32),
    )(g, b, res, mask, pfs, gfs, wo0, bo0, wo1, bo1, wo2, bo2)


def _wp9(W, C):
    return W.reshape(_L, C, C).transpose(1, 0, 2).reshape(C, _L * C)


def kernel(x, spiral_idx, W_point, W_r1a, b_r1a, W_r1b, b_r1b, W_mid,
           W_s0a, b_s0a, W_s0b, b_s0b, W_s1a, b_s1a, W_s1b, b_s1b,
           W_s2a, b_s2a, W_s2b, b_s2b, W_o0, b_o0, W_o1, b_o1, W_o2, b_o2):
    f32 = jnp.float32
    xp = jnp.pad(x, ((0, 0), (0, _VP - _V), (0, 1))).reshape(_N, -1)
    wp = jnp.pad(W_point, ((0, 1), (0, 0)))
    vmask = (jnp.arange(_VP) < _V).astype(f32)
    mask = jnp.tile(vmask, _B)[:, None]

    idx_pad = jnp.pad(spiral_idx, ((0, _VP - _V - 1), (0, 0)))
    offs = jnp.arange(_L, dtype=idx_pad.dtype)[None, :]
    gidx = jnp.concatenate(
        [(b * _VP + idx_pad) * _L + offs for b in range(_B)], axis=0
    )
    gidx = gidx.reshape(_NW, _VPW, _L).transpose(0, 2, 1).reshape(_NW, _L * _VPW)
    gidx = gidx.astype(jnp.int32)

    rb = lambda v: v.reshape(1, -1)
    _GS256 = _make_gather_sum(256, 32)
    _GS128 = _make_gather_sum(128, 64)

    pfs, Z = _k1(xp, wp, _wp9(W_r1a, 256), mask)
    g = _GS256(Z.reshape(_N * _L, 256), gidx)
    Z = _act_mm(g, rb(b_r1a), mask, _wp9(W_r1b, 256))
    g = _GS256(Z.reshape(_N * _L, 256), gidx)
    fs, Z = _mid(g, rb(b_r1b), pfs, mask, W_mid, _wp9(W_s0a, 128))
    gfs = _gmax(fs, mask)
    g = _GS128(Z.reshape(_N * _L, 128), gidx)
    Z = _act_mm(g, rb(b_s0a), mask, _wp9(W_s0b, 128))
    g = _GS128(Z.reshape(_N * _L, 128), gidx)
    fs, Z = _res_mm(g, rb(b_s0b), fs, mask, _wp9(W_s1a, 128))
    g = _GS128(Z.reshape(_N * _L, 128), gidx)
    Z = _act_mm(g, rb(b_s1a), mask, _wp9(W_s1b, 128))
    g = _GS128(Z.reshape(_N * _L, 128), gidx)
    fs, Z = _res_mm(g, rb(b_s1b), fs, mask, _wp9(W_s2a, 128))
    g = _GS128(Z.reshape(_N * _L, 128), gidx)
    Z = _act_mm(g, rb(b_s2a), mask, _wp9(W_s2b, 128))
    g = _GS128(Z.reshape(_N * _L, 128), gidx)
    o = _out_mlp(g, rb(b_s2b), fs, mask, pfs, gfs,
                 W_o0, rb(b_o0), W_o1, rb(b_o1), W_o2, rb(b_o2))
    return o.reshape(_B, _VP, 3)[:, :_V, :]


# speedup vs baseline: 5.5970x; 1.1332x over previous
import functools

import jax
import jax.numpy as jnp
from jax import lax
from jax.experimental import pallas as pl
from jax.experimental.pallas import tpu as pltpu
from jax.experimental.pallas import tpu_sc as plsc

_B = 2
_V = 5000
_L = 9
_VP = 5120
_N = _B * _VP
_BR = 512

_NC, _NS = 2, 16
_NW = _NC * _NS
_VPW = _N // _NW


@functools.lru_cache(maxsize=None)
def _make_gather_sum(C, CH):
    NCH = _VPW // CH
    nc16 = C // 16
    mesh = plsc.VectorSubcoreMesh(core_axis_name="c", subcore_axis_name="s")

    @functools.partial(
        pl.kernel,
        mesh=mesh,
        out_type=jax.ShapeDtypeStruct((_N, C), jnp.float32),
        scratch_types=[
            pltpu.VMEM((_L * _VPW,), jnp.int32),
            pltpu.VMEM((2, _L, CH, C), jnp.float32),
            pltpu.VMEM((CH, C), jnp.float32),
            pltpu.SemaphoreType.DMA,
            pltpu.SemaphoreType.DMA,
        ],
    )
    def gs(table_hbm, gidx_hbm, out_hbm, idx_v, buf_v, acc_v, sem0, sem1):
        wid = lax.axis_index("s") * _NC + lax.axis_index("c")
        pltpu.sync_copy(gidx_hbm.at[wid], idx_v)
        sems = (sem0, sem1)

        def fire(c):
            return [
                pltpu.async_copy(
                    table_hbm.at[idx_v.at[pl.ds(j * _VPW + c * CH, CH)]],
                    buf_v.at[c % 2, j],
                    sems[c % 2],
                )
                for j in range(_L)
            ]

        pend = fire(0)
        for c in range(NCH):
            for cp in pend:
                cp.wait()
            pend = fire(c + 1) if c + 1 < NCH else []

            def rbody(r, _, _p=c % 2):
                for cb in range(nc16):
                    co = cb * 16
                    s = buf_v[_p, 0, r, pl.ds(co, 16)]
                    for j in range(1, _L):
                        s = s + buf_v[_p, j, r, pl.ds(co, 16)]
                    acc_v[r, pl.ds(co, 16)] = s
                return 0

            lax.fori_loop(0, CH, rbody, 0)
            pltpu.sync_copy(acc_v, out_hbm.at[pl.ds(wid * _VPW + c * CH, CH)])

    return gs


def _k1_body(x_ref, wp_ref, w1_ref, m_ref, pfs_ref, z_ref):
    h = jnp.dot(x_ref[...], wp_ref[...], preferred_element_type=jnp.float32)
    h = jnp.maximum(h, 0.0) * m_ref[...]
    pfs_ref[...] = h
    z_ref[...] = jnp.dot(h, w1_ref[...], preferred_element_type=jnp.float32)


def _k1(xp, wp, w1, mask):
    K = xp.shape[1]
    CO = w1.shape[1]
    return pl.pallas_call(
        _k1_body,
        grid=(_N // _BR,),
        in_specs=[
            pl.BlockSpec((_BR, K), lambda i: (i, 0)),
            pl.BlockSpec((K, 256), lambda i: (0, 0)),
            pl.BlockSpec((256, CO), lambda i: (0, 0)),
            pl.BlockSpec((_BR, 1), lambda i: (i, 0)),
        ],
        out_specs=[
            pl.BlockSpec((_BR, 256), lambda i: (i, 0)),
            pl.BlockSpec((_BR, CO), lambda i: (i, 0)),
        ],
        out_shape=[
            jax.ShapeDtypeStruct((_N, 256), jnp.float32),
            jax.ShapeDtypeStruct((_N, CO), jnp.float32),
        ],
    )(xp, wp, w1, mask)


def _act_mm_body(g_ref, b_ref, m_ref, w_ref, z_ref):
    h = jnp.maximum(g_ref[...] + b_ref[...], 0.0) * m_ref[...]
    z_ref[...] = jnp.dot(h, w_ref[...], preferred_element_type=jnp.float32)


def _act_mm(g, b, mask, w):
    C = g.shape[1]
    CO = w.shape[1]
    return pl.pallas_call(
        _act_mm_body,
        grid=(_N // _BR,),
        in_specs=[
            pl.BlockSpec((_BR, C), lambda i: (i, 0)),
            pl.BlockSpec((1, C), lambda i: (0, 0)),
            pl.BlockSpec((_BR, 1), lambda i: (i, 0)),
            pl.BlockSpec((C, CO), lambda i: (0, 0)),
        ],
        out_specs=pl.BlockSpec((_BR, CO), lambda i: (i, 0)),
        out_shape=jax.ShapeDtypeStruct((_N, CO), jnp.float32),
    )(g, b, mask, w)


def _res_mm_body(g_ref, b_ref, r_ref, m_ref, w_ref, fs_ref, z_ref):
    h = jnp.maximum(g_ref[...] + b_ref[...] + r_ref[...], 0.0) * m_ref[...]
    fs_ref[...] = h
    z_ref[...] = jnp.dot(h, w_ref[...], preferred_element_type=jnp.float32)


def _res_mm(g, b, res, mask, w):
    C = g.shape[1]
    CO = w.shape[1]
    return pl.pallas_call(
        _res_mm_body,
        grid=(_N // _BR,),
        in_specs=[
            pl.BlockSpec((_BR, C), lambda i: (i, 0)),
            pl.BlockSpec((1, C), lambda i: (0, 0)),
            pl.BlockSpec((_BR, C), lambda i: (i, 0)),
            pl.BlockSpec((_BR, 1), lambda i: (i, 0)),
            pl.BlockSpec((C, CO), lambda i: (0, 0)),
        ],
        out_specs=[
            pl.BlockSpec((_BR, C), lambda i: (i, 0)),
            pl.BlockSpec((_BR, CO), lambda i: (i, 0)),
        ],
        out_shape=[
            jax.ShapeDtypeStruct((_N, C), jnp.float32),
            jax.ShapeDtypeStruct((_N, CO), jnp.float32),
        ],
    )(g, b, res, mask, w)


def _mid_body(g_ref, b_ref, r_ref, m_ref, wmid_ref, wn_ref, fs_ref, z_ref):
    h = jnp.maximum(g_ref[...] + b_ref[...] + r_ref[...], 0.0) * m_ref[...]
    fs = jnp.dot(h, wmid_ref[...], preferred_element_type=jnp.float32)
    fs_ref[...] = fs
    z_ref[...] = jnp.dot(fs, wn_ref[...], preferred_element_type=jnp.float32)


def _mid(g, b, res, mask, wmid, wn):
    CO = wn.shape[1]
    return pl.pallas_call(
        _mid_body,
        grid=(_N // _BR,),
        in_specs=[
            pl.BlockSpec((_BR, 256), lambda i: (i, 0)),
            pl.BlockSpec((1, 256), lambda i: (0, 0)),
            pl.BlockSpec((_BR, 256), lambda i: (i, 0)),
            pl.BlockSpec((_BR, 1), lambda i: (i, 0)),
            pl.BlockSpec((256, 128), lambda i: (0, 0)),
            pl.BlockSpec((128, CO), lambda i: (0, 0)),
        ],
        out_specs=[
            pl.BlockSpec((_BR, 128), lambda i: (i, 0)),
            pl.BlockSpec((_BR, CO), lambda i: (i, 0)),
        ],
        out_shape=[
            jax.ShapeDtypeStruct((_N, 128), jnp.float32),
            jax.ShapeDtypeStruct((_N, CO), jnp.float32),
        ],
    )(g, b, res, mask, wmid, wn)


def _max_body(fs_ref, m_ref, o_ref):
    m = jnp.where(m_ref[...] > 0.0, fs_ref[...], -1e30)
    o_ref[...] = jnp.max(m.reshape(_B, _VP, 128), axis=1)


def _gmax(fs, mask):
    return pl.pallas_call(
        _max_body,
        in_specs=[
            pl.BlockSpec((_N, 128), lambda: (0, 0)),
            pl.BlockSpec((_N, 1), lambda: (0, 0)),
        ],
        out_specs=pl.BlockSpec((_B, 128), lambda: (0, 0)),
        out_shape=jax.ShapeDtypeStruct((_B, 128), jnp.float32),
    )(fs, mask)


def _out_body(g_ref, b_ref, r_ref, m_ref, pfs_ref, gfs_ref,
              wo0_ref, bo0_ref, wo1_ref, bo1_ref, wo2_ref, bo2_ref, o_ref):
    fs3 = jnp.maximum(g_ref[...] + b_ref[...] + r_ref[...], 0.0) * m_ref[...]
    bsel = lax.broadcasted_iota(jnp.int32, (_B, 128), 0) == pl.program_id(0)
    grow = jnp.sum(jnp.where(bsel, gfs_ref[...], 0.0), axis=0, keepdims=True)
    t = jnp.dot(pfs_ref[...], wo0_ref[0:256], preferred_element_type=jnp.float32)
    t = t + jnp.dot(fs3, wo0_ref[256:384], preferred_element_type=jnp.float32)
    t = t + jnp.dot(grow, wo0_ref[384:512], preferred_element_type=jnp.float32)
    o = jnp.maximum(t + bo0_ref[...], 0.0)
    o = jnp.maximum(
        jnp.dot(o, wo1_ref[...], preferred_element_type=jnp.float32) + bo1_ref[...], 0.0
    )
    o_ref[...] = jnp.dot(o, wo2_ref[...], preferred_element_type=jnp.float32) + bo2_ref[...]


def _out_mlp(g, b, res, mask, pfs, gfs, wo0, bo0, wo1, bo1, wo2, bo2):
    nrb = _VP // _BR
    row = lambda b_, i: (b_ * nrb + i, 0)
    return pl.pallas_call(
        _out_body,
        grid=(_B, nrb),
        in_specs=[
            pl.BlockSpec((_BR, 128), row),
            pl.BlockSpec((1, 128), lambda b_, i: (0, 0)),
            pl.BlockSpec((_BR, 128), row),
            pl.BlockSpec((_BR, 1), row),
            pl.BlockSpec((_BR, 256), row),
            pl.BlockSpec((_B, 128), lambda b_, i: (0, 0)),
            pl.BlockSpec((512, 256), lambda b_, i: (0, 0)),
            pl.BlockSpec((1, 256), lambda b_, i: (0, 0)),
            pl.BlockSpec((256, 128), lambda b_, i: (0, 0)),
            pl.BlockSpec((1, 128), lambda b_, i: (0, 0)),
            pl.BlockSpec((128, 3), lambda b_, i: (0, 0)),
            pl.BlockSpec((1, 3), lambda b_, i: (0, 0)),
        ],
        out_specs=pl.BlockSpec((_BR, 3), row),
        out_shape=jax.ShapeDtypeStruct((_N, 3), jnp.float32),
    )(g, b, res, mask, pfs, gfs, wo0, bo0, wo1, bo1, wo2, bo2)


def _wp9(W, C):
    return W.reshape(_L, C, C).transpose(1, 0, 2).reshape(C, _L * C)


def kernel(x, spiral_idx, W_point, W_r1a, b_r1a, W_r1b, b_r1b, W_mid,
           W_s0a, b_s0a, W_s0b, b_s0b, W_s1a, b_s1a, W_s1b, b_s1b,
           W_s2a, b_s2a, W_s2b, b_s2b, W_o0, b_o0, W_o1, b_o1, W_o2, b_o2):
    f32 = jnp.float32
    xp = jnp.pad(x, ((0, 0), (0, _VP - _V), (0, 1))).reshape(_N, -1)
    wp = jnp.pad(W_point, ((0, 1), (0, 0)))
    vmask = (jnp.arange(_VP) < _V).astype(f32)
    mask = jnp.tile(vmask, _B)[:, None]

    idx_pad = jnp.pad(spiral_idx, ((0, _VP - _V - 1), (0, 0)))
    offs = jnp.arange(_L, dtype=idx_pad.dtype)[None, :]
    gidx = jnp.concatenate(
        [(b * _VP + idx_pad) * _L + offs for b in range(_B)], axis=0
    )
    gidx = gidx.reshape(_NW, _VPW, _L).transpose(0, 2, 1).reshape(_NW, _L * _VPW)
    gidx = gidx.astype(jnp.int32)

    rb = lambda v: v.reshape(1, -1)
    _GS256 = _make_gather_sum(256, 16)
    _GS128 = _make_gather_sum(128, 32)

    pfs, Z = _k1(xp, wp, _wp9(W_r1a, 256), mask)
    g = _GS256(Z.reshape(_N * _L, 256), gidx)
    Z = _act_mm(g, rb(b_r1a), mask, _wp9(W_r1b, 256))
    g = _GS256(Z.reshape(_N * _L, 256), gidx)
    fs, Z = _mid(g, rb(b_r1b), pfs, mask, W_mid, _wp9(W_s0a, 128))
    gfs = _gmax(fs, mask)
    g = _GS128(Z.reshape(_N * _L, 128), gidx)
    Z = _act_mm(g, rb(b_s0a), mask, _wp9(W_s0b, 128))
    g = _GS128(Z.reshape(_N * _L, 128), gidx)
    fs, Z = _res_mm(g, rb(b_s0b), fs, mask, _wp9(W_s1a, 128))
    g = _GS128(Z.reshape(_N * _L, 128), gidx)
    Z = _act_mm(g, rb(b_s1a), mask, _wp9(W_s1b, 128))
    g = _GS128(Z.reshape(_N * _L, 128), gidx)
    fs, Z = _res_mm(g, rb(b_s1b), fs, mask, _wp9(W_s2a, 128))
    g = _GS128(Z.reshape(_N * _L, 128), gidx)
    Z = _act_mm(g, rb(b_s2a), mask, _wp9(W_s2b, 128))
    g = _GS128(Z.reshape(_N * _L, 128), gidx)
    o = _out_mlp(g, rb(b_s2b), fs, mask, pfs, gfs,
                 W_o0, rb(b_o0), W_o1, rb(b_o1), W_o2, rb(b_o2))
    return o.reshape(_B, _VP, 3)[:, :_V, :]


# speedup vs baseline: 8.7480x; 1.5630x over previous
import functools

import jax
import jax.numpy as jnp
from jax import lax
from jax.experimental import pallas as pl
from jax.experimental.pallas import tpu as pltpu
from jax.experimental.pallas import tpu_sc as plsc

_B = 2
_V = 5000
_L = 9
_VP = 5120
_N = _B * _VP
_BR = 512

_NC, _NS = 2, 16
_NW = _NC * _NS
_VPW = _N // _NW


@functools.lru_cache(maxsize=None)
def _make_gather_sum(C, CH):
    NCH = _VPW // CH
    nc16 = C // 16
    mesh = plsc.VectorSubcoreMesh(core_axis_name="c", subcore_axis_name="s")

    @functools.partial(
        pl.kernel,
        mesh=mesh,
        out_type=jax.ShapeDtypeStruct((_N, C), jnp.float32),
        scratch_types=[
            pltpu.VMEM((_L * _VPW,), jnp.int32),
            pltpu.VMEM((2, _L, CH, C), jnp.float32),
            pltpu.VMEM((CH, C), jnp.float32),
            pltpu.SemaphoreType.DMA,
            pltpu.SemaphoreType.DMA,
        ],
    )
    def gs(table_hbm, gidx_hbm, out_hbm, idx_v, buf_v, acc_v, sem0, sem1):
        wid = lax.axis_index("s") * _NC + lax.axis_index("c")
        pltpu.sync_copy(gidx_hbm.at[wid], idx_v)
        sems = (sem0, sem1)

        def fire(c):
            return [
                pltpu.async_copy(
                    table_hbm.at[idx_v.at[pl.ds(j * _VPW + c * CH, CH)]],
                    buf_v.at[c % 2, j],
                    sems[c % 2],
                )
                for j in range(_L)
            ]

        pend = fire(0)
        for c in range(NCH):
            for cp in pend:
                cp.wait()
            pend = fire(c + 1) if c + 1 < NCH else []

            def rbody(r, _, _p=c % 2):
                for cb in range(nc16):
                    co = cb * 16
                    s = buf_v[_p, 0, r, pl.ds(co, 16)]
                    for j in range(1, _L):
                        s = s + buf_v[_p, j, r, pl.ds(co, 16)]
                    acc_v[r, pl.ds(co, 16)] = s
                return 0

            lax.fori_loop(0, CH, rbody, 0)
            pltpu.sync_copy(acc_v, out_hbm.at[pl.ds(wid * _VPW + c * CH, CH)])

    return gs


def _k1_body(x_ref, wp_ref, w1_ref, m_ref, pfs_ref, z_ref):
    h = jnp.dot(x_ref[...], wp_ref[...], preferred_element_type=jnp.float32)
    h = jnp.maximum(h, 0.0) * m_ref[...]
    pfs_ref[...] = h
    for j in range(_L):
        z_ref[j] = jnp.dot(h, w1_ref[j], preferred_element_type=jnp.float32)


def _k1(xp, wp, w1, mask):
    K = xp.shape[1]
    C = w1.shape[2]
    return pl.pallas_call(
        _k1_body,
        grid=(_N // _BR,),
        in_specs=[
            pl.BlockSpec((_BR, K), lambda i: (i, 0)),
            pl.BlockSpec((K, 256), lambda i: (0, 0)),
            pl.BlockSpec((_L, 256, C), lambda i: (0, 0, 0)),
            pl.BlockSpec((_BR, 1), lambda i: (i, 0)),
        ],
        out_specs=[
            pl.BlockSpec((_BR, 256), lambda i: (i, 0)),
            pl.BlockSpec((_L, _BR, C), lambda i: (0, i, 0)),
        ],
        out_shape=[
            jax.ShapeDtypeStruct((_N, 256), jnp.float32),
            jax.ShapeDtypeStruct((_L, _N, C), jnp.float32),
        ],
    )(xp, wp, w1, mask)


def _act_mm_body(g_ref, b_ref, m_ref, w_ref, z_ref):
    h = jnp.maximum(g_ref[...] + b_ref[...], 0.0) * m_ref[...]
    for j in range(_L):
        z_ref[j] = jnp.dot(h, w_ref[j], preferred_element_type=jnp.float32)


def _act_mm(g, b, mask, w):
    C = g.shape[1]
    CO = w.shape[2]
    return pl.pallas_call(
        _act_mm_body,
        grid=(_N // _BR,),
        in_specs=[
            pl.BlockSpec((_BR, C), lambda i: (i, 0)),
            pl.BlockSpec((1, C), lambda i: (0, 0)),
            pl.BlockSpec((_BR, 1), lambda i: (i, 0)),
            pl.BlockSpec((_L, C, CO), lambda i: (0, 0, 0)),
        ],
        out_specs=pl.BlockSpec((_L, _BR, CO), lambda i: (0, i, 0)),
        out_shape=jax.ShapeDtypeStruct((_L, _N, CO), jnp.float32),
    )(g, b, mask, w)


def _res_mm_body(g_ref, b_ref, r_ref, m_ref, w_ref, fs_ref, z_ref):
    h = jnp.maximum(g_ref[...] + b_ref[...] + r_ref[...], 0.0) * m_ref[...]
    fs_ref[...] = h
    for j in range(_L):
        z_ref[j] = jnp.dot(h, w_ref[j], preferred_element_type=jnp.float32)


def _res_mm(g, b, res, mask, w):
    C = g.shape[1]
    CO = w.shape[2]
    return pl.pallas_call(
        _res_mm_body,
        grid=(_N // _BR,),
        in_specs=[
            pl.BlockSpec((_BR, C), lambda i: (i, 0)),
            pl.BlockSpec((1, C), lambda i: (0, 0)),
            pl.BlockSpec((_BR, C), lambda i: (i, 0)),
            pl.BlockSpec((_BR, 1), lambda i: (i, 0)),
            pl.BlockSpec((_L, C, CO), lambda i: (0, 0, 0)),
        ],
        out_specs=[
            pl.BlockSpec((_BR, C), lambda i: (i, 0)),
            pl.BlockSpec((_L, _BR, CO), lambda i: (0, i, 0)),
        ],
        out_shape=[
            jax.ShapeDtypeStruct((_N, C), jnp.float32),
            jax.ShapeDtypeStruct((_L, _N, CO), jnp.float32),
        ],
    )(g, b, res, mask, w)


def _mid_body(g_ref, b_ref, r_ref, m_ref, wmid_ref, wn_ref, fs_ref, z_ref):
    h = jnp.maximum(g_ref[...] + b_ref[...] + r_ref[...], 0.0) * m_ref[...]
    fs = jnp.dot(h, wmid_ref[...], preferred_element_type=jnp.float32)
    fs_ref[...] = fs
    for j in range(_L):
        z_ref[j] = jnp.dot(fs, wn_ref[j], preferred_element_type=jnp.float32)


def _mid(g, b, res, mask, wmid, wn):
    CO = wn.shape[2]
    return pl.pallas_call(
        _mid_body,
        grid=(_N // _BR,),
        in_specs=[
            pl.BlockSpec((_BR, 256), lambda i: (i, 0)),
            pl.BlockSpec((1, 256), lambda i: (0, 0)),
            pl.BlockSpec((_BR, 256), lambda i: (i, 0)),
            pl.BlockSpec((_BR, 1), lambda i: (i, 0)),
            pl.BlockSpec((256, 128), lambda i: (0, 0)),
            pl.BlockSpec((_L, 128, CO), lambda i: (0, 0, 0)),
        ],
        out_specs=[
            pl.BlockSpec((_BR, 128), lambda i: (i, 0)),
            pl.BlockSpec((_L, _BR, CO), lambda i: (0, i, 0)),
        ],
        out_shape=[
            jax.ShapeDtypeStruct((_N, 128), jnp.float32),
            jax.ShapeDtypeStruct((_L, _N, CO), jnp.float32),
        ],
    )(g, b, res, mask, wmid, wn)


def _max_body(fs_ref, m_ref, o_ref):
    m = jnp.where(m_ref[...] > 0.0, fs_ref[...], -1e30)
    o_ref[...] = jnp.max(m.reshape(_B, _VP, 128), axis=1)


def _gmax(fs, mask):
    return pl.pallas_call(
        _max_body,
        in_specs=[
            pl.BlockSpec((_N, 128), lambda: (0, 0)),
            pl.BlockSpec((_N, 1), lambda: (0, 0)),
        ],
        out_specs=pl.BlockSpec((_B, 128), lambda: (0, 0)),
        out_shape=jax.ShapeDtypeStruct((_B, 128), jnp.float32),
    )(fs, mask)


def _out_body(g_ref, b_ref, r_ref, m_ref, pfs_ref, gfs_ref,
              wo0_ref, bo0_ref, wo1_ref, bo1_ref, wo2_ref, bo2_ref, o_ref):
    fs3 = jnp.maximum(g_ref[...] + b_ref[...] + r_ref[...], 0.0) * m_ref[...]
    bsel = lax.broadcasted_iota(jnp.int32, (_B, 128), 0) == pl.program_id(0)
    grow = jnp.sum(jnp.where(bsel, gfs_ref[...], 0.0), axis=0, keepdims=True)
    t = jnp.dot(pfs_ref[...], wo0_ref[0:256], preferred_element_type=jnp.float32)
    t = t + jnp.dot(fs3, wo0_ref[256:384], preferred_element_type=jnp.float32)
    t = t + jnp.dot(grow, wo0_ref[384:512], preferred_element_type=jnp.float32)
    o = jnp.maximum(t + bo0_ref[...], 0.0)
    o = jnp.maximum(
        jnp.dot(o, wo1_ref[...], preferred_element_type=jnp.float32) + bo1_ref[...], 0.0
    )
    o_ref[...] = jnp.dot(o, wo2_ref[...], preferred_element_type=jnp.float32) + bo2_ref[...]


def _out_mlp(g, b, res, mask, pfs, gfs, wo0, bo0, wo1, bo1, wo2, bo2):
    nrb = _VP // _BR
    row = lambda b_, i: (b_ * nrb + i, 0)
    return pl.pallas_call(
        _out_body,
        grid=(_B, nrb),
        in_specs=[
            pl.BlockSpec((_BR, 128), row),
            pl.BlockSpec((1, 128), lambda b_, i: (0, 0)),
            pl.BlockSpec((_BR, 128), row),
            pl.BlockSpec((_BR, 1), row),
            pl.BlockSpec((_BR, 256), row),
            pl.BlockSpec((_B, 128), lambda b_, i: (0, 0)),
            pl.BlockSpec((512, 256), lambda b_, i: (0, 0)),
            pl.BlockSpec((1, 256), lambda b_, i: (0, 0)),
            pl.BlockSpec((256, 128), lambda b_, i: (0, 0)),
            pl.BlockSpec((1, 128), lambda b_, i: (0, 0)),
            pl.BlockSpec((128, 3), lambda b_, i: (0, 0)),
            pl.BlockSpec((1, 3), lambda b_, i: (0, 0)),
        ],
        out_specs=pl.BlockSpec((_BR, 3), row),
        out_shape=jax.ShapeDtypeStruct((_N, 3), jnp.float32),
    )(g, b, res, mask, pfs, gfs, wo0, bo0, wo1, bo1, wo2, bo2)


def _w9(W, C):
    return W.reshape(_L, C, C)


def kernel(x, spiral_idx, W_point, W_r1a, b_r1a, W_r1b, b_r1b, W_mid,
           W_s0a, b_s0a, W_s0b, b_s0b, W_s1a, b_s1a, W_s1b, b_s1b,
           W_s2a, b_s2a, W_s2b, b_s2b, W_o0, b_o0, W_o1, b_o1, W_o2, b_o2):
    f32 = jnp.float32
    xp = jnp.pad(x, ((0, 0), (0, _VP - _V), (0, 1))).reshape(_N, -1)
    wp = jnp.pad(W_point, ((0, 1), (0, 0)))
    vmask = (jnp.arange(_VP) < _V).astype(f32)
    mask = jnp.tile(vmask, _B)[:, None]

    idx_pad = jnp.pad(spiral_idx, ((0, _VP - _V - 1), (0, 0)))
    offs = jnp.arange(_L, dtype=idx_pad.dtype)[None, :] * _N
    gidx = jnp.concatenate(
        [(b * _VP + idx_pad) + offs for b in range(_B)], axis=0
    )
    gidx = gidx.reshape(_NW, _VPW, _L).transpose(0, 2, 1).reshape(_NW, _L * _VPW)
    gidx = gidx.astype(jnp.int32)

    rb = lambda v: v.reshape(1, -1)
    _GS256 = _make_gather_sum(256, 16)
    _GS128 = _make_gather_sum(128, 32)

    pfs, Z = _k1(xp, wp, _w9(W_r1a, 256), mask)
    g = _GS256(Z.reshape(_N * _L, 256), gidx)
    Z = _act_mm(g, rb(b_r1a), mask, _w9(W_r1b, 256))
    g = _GS256(Z.reshape(_N * _L, 256), gidx)
    fs, Z = _mid(g, rb(b_r1b), pfs, mask, W_mid, _w9(W_s0a, 128))
    gfs = _gmax(fs, mask)
    g = _GS128(Z.reshape(_N * _L, 128), gidx)
    Z = _act_mm(g, rb(b_s0a), mask, _w9(W_s0b, 128))
    g = _GS128(Z.reshape(_N * _L, 128), gidx)
    fs, Z = _res_mm(g, rb(b_s0b), fs, mask, _w9(W_s1a, 128))
    g = _GS128(Z.reshape(_N * _L, 128), gidx)
    Z = _act_mm(g, rb(b_s1a), mask, _w9(W_s1b, 128))
    g = _GS128(Z.reshape(_N * _L, 128), gidx)
    fs, Z = _res_mm(g, rb(b_s1b), fs, mask, _w9(W_s2a, 128))
    g = _GS128(Z.reshape(_N * _L, 128), gidx)
    Z = _act_mm(g, rb(b_s2a), mask, _w9(W_s2b, 128))
    g = _GS128(Z.reshape(_N * _L, 128), gidx)
    o = _out_mlp(g, rb(b_s2b), fs, mask, pfs, gfs,
                 W_o0, rb(b_o0), W_o1, rb(b_o1), W_o2, rb(b_o2))
    return o.reshape(_B, _VP, 3)[:, :_V, :]


# speedup vs baseline: 9.5456x; 1.0912x over previous
import functools

import jax
import jax.numpy as jnp
from jax import lax
from jax.experimental import pallas as pl
from jax.experimental.pallas import tpu as pltpu
from jax.experimental.pallas import tpu_sc as plsc

_B = 2
_V = 5000
_L = 9
_VP = 5120
_BR = 512
_NRB = _VP // _BR

_NC, _NS = 2, 16
_NW = _NC * _NS
_VPW = _VP // _NW


@functools.lru_cache(maxsize=None)
def _make_gather_sum(C, CH):
    NCH = _VPW // CH
    nc16 = C // 16
    mesh = plsc.VectorSubcoreMesh(core_axis_name="c", subcore_axis_name="s")

    @functools.partial(
        pl.kernel,
        mesh=mesh,
        out_type=jax.ShapeDtypeStruct((_VP, C), jnp.float32),
        scratch_types=[
            pltpu.VMEM((_L * _VPW,), jnp.int32),
            pltpu.VMEM((2, _L, CH, C), jnp.float32),
            pltpu.VMEM((CH, C), jnp.float32),
            pltpu.SemaphoreType.DMA,
            pltpu.SemaphoreType.DMA,
        ],
    )
    def gs(table_hbm, gidx_hbm, out_hbm, idx_v, buf_v, acc_v, sem0, sem1):
        wid = lax.axis_index("s") * _NC + lax.axis_index("c")
        pltpu.sync_copy(gidx_hbm.at[wid], idx_v)
        sems = (sem0, sem1)

        def fire(c):
            return [
                pltpu.async_copy(
                    table_hbm.at[idx_v.at[pl.ds(j * _VPW + c * CH, CH)]],
                    buf_v.at[c % 2, j],
                    sems[c % 2],
                )
                for j in range(_L)
            ]

        pend = fire(0)
        for c in range(NCH):
            for cp in pend:
                cp.wait()
            pend = fire(c + 1) if c + 1 < NCH else []

            def rbody(r, _, _p=c % 2):
                for cb in range(nc16):
                    co = cb * 16
                    s = buf_v[_p, 0, r, pl.ds(co, 16)]
                    for j in range(1, _L):
                        s = s + buf_v[_p, j, r, pl.ds(co, 16)]
                    acc_v[r, pl.ds(co, 16)] = s
                return 0

            lax.fori_loop(0, CH, rbody, 0)
            pltpu.sync_copy(acc_v, out_hbm.at[pl.ds(wid * _VPW + c * CH, CH)])

    return gs


def _k1_body(x_ref, wp_ref, w1_ref, m_ref, pfs_ref, z_ref):
    h = jnp.dot(x_ref[...], wp_ref[...], preferred_element_type=jnp.float32)
    h = jnp.maximum(h, 0.0) * m_ref[...]
    pfs_ref[...] = h
    for j in range(_L):
        z_ref[j] = jnp.dot(h, w1_ref[j], preferred_element_type=jnp.float32)


def _k1(xp, wp, w1, mask):
    K = xp.shape[1]
    C = w1.shape[2]
    return pl.pallas_call(
        _k1_body,
        grid=(_NRB,),
        in_specs=[
            pl.BlockSpec((_BR, K), lambda i: (i, 0)),
            pl.BlockSpec((K, 256), lambda i: (0, 0)),
            pl.BlockSpec((_L, 256, C), lambda i: (0, 0, 0)),
            pl.BlockSpec((_BR, 1), lambda i: (i, 0)),
        ],
        out_specs=[
            pl.BlockSpec((_BR, 256), lambda i: (i, 0)),
            pl.BlockSpec((_L, _BR, C), lambda i: (0, i, 0)),
        ],
        out_shape=[
            jax.ShapeDtypeStruct((_VP, 256), jnp.float32),
            jax.ShapeDtypeStruct((_L, _VP, C), jnp.float32),
        ],
    )(xp, wp, w1, mask)


def _act_mm_body(g_ref, b_ref, m_ref, w_ref, z_ref):
    h = jnp.maximum(g_ref[...] + b_ref[...], 0.0) * m_ref[...]
    for j in range(_L):
        z_ref[j] = jnp.dot(h, w_ref[j], preferred_element_type=jnp.float32)


def _act_mm(g, b, mask, w):
    C = g.shape[1]
    CO = w.shape[2]
    return pl.pallas_call(
        _act_mm_body,
        grid=(_NRB,),
        in_specs=[
            pl.BlockSpec((_BR, C), lambda i: (i, 0)),
            pl.BlockSpec((1, C), lambda i: (0, 0)),
            pl.BlockSpec((_BR, 1), lambda i: (i, 0)),
            pl.BlockSpec((_L, C, CO), lambda i: (0, 0, 0)),
        ],
        out_specs=pl.BlockSpec((_L, _BR, CO), lambda i: (0, i, 0)),
        out_shape=jax.ShapeDtypeStruct((_L, _VP, CO), jnp.float32),
    )(g, b, mask, w)


def _res_mm_body(g_ref, b_ref, r_ref, m_ref, w_ref, fs_ref, z_ref):
    h = jnp.maximum(g_ref[...] + b_ref[...] + r_ref[...], 0.0) * m_ref[...]
    fs_ref[...] = h
    for j in range(_L):
        z_ref[j] = jnp.dot(h, w_ref[j], preferred_element_type=jnp.float32)


def _res_mm(g, b, res, mask, w):
    C = g.shape[1]
    CO = w.shape[2]
    return pl.pallas_call(
        _res_mm_body,
        grid=(_NRB,),
        in_specs=[
            pl.BlockSpec((_BR, C), lambda i: (i, 0)),
            pl.BlockSpec((1, C), lambda i: (0, 0)),
            pl.BlockSpec((_BR, C), lambda i: (i, 0)),
            pl.BlockSpec((_BR, 1), lambda i: (i, 0)),
            pl.BlockSpec((_L, C, CO), lambda i: (0, 0, 0)),
        ],
        out_specs=[
            pl.BlockSpec((_BR, C), lambda i: (i, 0)),
            pl.BlockSpec((_L, _BR, CO), lambda i: (0, i, 0)),
        ],
        out_shape=[
            jax.ShapeDtypeStruct((_VP, C), jnp.float32),
            jax.ShapeDtypeStruct((_L, _VP, CO), jnp.float32),
        ],
    )(g, b, res, mask, w)


def _mid_body(g_ref, b_ref, r_ref, m_ref, wmid_ref, wn_ref, fs_ref, z_ref):
    h = jnp.maximum(g_ref[...] + b_ref[...] + r_ref[...], 0.0) * m_ref[...]
    fs = jnp.dot(h, wmid_ref[...], preferred_element_type=jnp.float32)
    fs_ref[...] = fs
    for j in range(_L):
        z_ref[j] = jnp.dot(fs, wn_ref[j], preferred_element_type=jnp.float32)


def _mid(g, b, res, mask, wmid, wn):
    CO = wn.shape[2]
    return pl.pallas_call(
        _mid_body,
        grid=(_NRB,),
        in_specs=[
            pl.BlockSpec((_BR, 256), lambda i: (i, 0)),
            pl.BlockSpec((1, 256), lambda i: (0, 0)),
            pl.BlockSpec((_BR, 256), lambda i: (i, 0)),
            pl.BlockSpec((_BR, 1), lambda i: (i, 0)),
            pl.BlockSpec((256, 128), lambda i: (0, 0)),
            pl.BlockSpec((_L, 128, CO), lambda i: (0, 0, 0)),
        ],
        out_specs=[
            pl.BlockSpec((_BR, 128), lambda i: (i, 0)),
            pl.BlockSpec((_L, _BR, CO), lambda i: (0, i, 0)),
        ],
        out_shape=[
            jax.ShapeDtypeStruct((_VP, 128), jnp.float32),
            jax.ShapeDtypeStruct((_L, _VP, CO), jnp.float32),
        ],
    )(g, b, res, mask, wmid, wn)


def _max_body(fs_ref, m_ref, o_ref):
    o_ref[...] = jnp.max(
        jnp.where(m_ref[...] > 0.0, fs_ref[...], -1e30), axis=0, keepdims=True
    )


def _gmax(fs, mask):
    return pl.pallas_call(
        _max_body,
        in_specs=[
            pl.BlockSpec((_VP, 128), lambda: (0, 0)),
            pl.BlockSpec((_VP, 1), lambda: (0, 0)),
        ],
        out_specs=pl.BlockSpec((1, 128), lambda: (0, 0)),
        out_shape=jax.ShapeDtypeStruct((1, 128), jnp.float32),
    )(fs, mask)


def _out_body(g_ref, b_ref, r_ref, m_ref, pfs_ref, gfs_ref,
              wo0_ref, bo0_ref, wo1_ref, bo1_ref, wo2_ref, bo2_ref, o_ref):
    fs3 = jnp.maximum(g_ref[...] + b_ref[...] + r_ref[...], 0.0) * m_ref[...]
    t = jnp.dot(pfs_ref[...], wo0_ref[0:256], preferred_element_type=jnp.float32)
    t = t + jnp.dot(fs3, wo0_ref[256:384], preferred_element_type=jnp.float32)
    t = t + jnp.dot(gfs_ref[...], wo0_ref[384:512], preferred_element_type=jnp.float32)
    o = jnp.maximum(t + bo0_ref[...], 0.0)
    o = jnp.maximum(
        jnp.dot(o, wo1_ref[...], preferred_element_type=jnp.float32) + bo1_ref[...], 0.0
    )
    o_ref[...] = jnp.dot(o, wo2_ref[...], preferred_element_type=jnp.float32) + bo2_ref[...]


def _out_mlp(g, b, res, mask, pfs, gfs, wo0, bo0, wo1, bo1, wo2, bo2):
    row = lambda i: (i, 0)
    full = lambda i: (0, 0)
    return pl.pallas_call(
        _out_body,
        grid=(_NRB,),
        in_specs=[
            pl.BlockSpec((_BR, 128), row),
            pl.BlockSpec((1, 128), full),
            pl.BlockSpec((_BR, 128), row),
            pl.BlockSpec((_BR, 1), row),
            pl.BlockSpec((_BR, 256), row),
            pl.BlockSpec((1, 128), full),
            pl.BlockSpec((512, 256), full),
            pl.BlockSpec((1, 256), full),
            pl.BlockSpec((256, 128), full),
            pl.BlockSpec((1, 128), full),
            pl.BlockSpec((128, 3), full),
            pl.BlockSpec((1, 3), full),
        ],
        out_specs=pl.BlockSpec((_BR, 3), row),
        out_shape=jax.ShapeDtypeStruct((_VP, 3), jnp.float32),
    )(g, b, res, mask, pfs, gfs, wo0, bo0, wo1, bo1, wo2, bo2)


def _w9(W, C):
    return W.reshape(_L, C, C)


def kernel(x, spiral_idx, W_point, W_r1a, b_r1a, W_r1b, b_r1b, W_mid,
           W_s0a, b_s0a, W_s0b, b_s0b, W_s1a, b_s1a, W_s1b, b_s1b,
           W_s2a, b_s2a, W_s2b, b_s2b, W_o0, b_o0, W_o1, b_o1, W_o2, b_o2):
    f32 = jnp.float32
    wp = jnp.pad(W_point, ((0, 1), (0, 0)))
    mask = (jnp.arange(_VP) < _V).astype(f32)[:, None]

    idx_pad = jnp.pad(spiral_idx, ((0, _VP - _V - 1), (0, 0)))
    offs = jnp.arange(_L, dtype=idx_pad.dtype)[None, :] * _VP
    gidx = idx_pad + offs
    gidx = gidx.reshape(_NW, _VPW, _L).transpose(0, 2, 1).reshape(_NW, _L * _VPW)
    gidx = gidx.astype(jnp.int32)

    rb = lambda v: v.reshape(1, -1)
    GS256 = _make_gather_sum(256, 16)
    GS128 = _make_gather_sum(128, 32)
    w_r1a, w_r1b = _w9(W_r1a, 256), _w9(W_r1b, 256)
    w_s0a, w_s0b = _w9(W_s0a, 128), _w9(W_s0b, 128)
    w_s1a, w_s1b = _w9(W_s1a, 128), _w9(W_s1b, 128)
    w_s2a, w_s2b = _w9(W_s2a, 128), _w9(W_s2b, 128)

    def chain(xb):
        xp = jnp.pad(xb, ((0, _VP - _V), (0, 1)))
        pfs, Z = _k1(xp, wp, w_r1a, mask)
        g = GS256(Z.reshape(_L * _VP, 256), gidx)
        Z = _act_mm(g, rb(b_r1a), mask, w_r1b)
        g = GS256(Z.reshape(_L * _VP, 256), gidx)
        fs, Z = _mid(g, rb(b_r1b), pfs, mask, W_mid, w_s0a)
        gfs = _gmax(fs, mask)
        g = GS128(Z.reshape(_L * _VP, 128), gidx)
        Z = _act_mm(g, rb(b_s0a), mask, w_s0b)
        g = GS128(Z.reshape(_L * _VP, 128), gidx)
        fs, Z = _res_mm(g, rb(b_s0b), fs, mask, w_s1a)
        g = GS128(Z.reshape(_L * _VP, 128), gidx)
        Z = _act_mm(g, rb(b_s1a), mask, w_s1b)
        g = GS128(Z.reshape(_L * _VP, 128), gidx)
        fs, Z = _res_mm(g, rb(b_s1b), fs, mask, w_s2a)
        g = GS128(Z.reshape(_L * _VP, 128), gidx)
        Z = _act_mm(g, rb(b_s2a), mask, w_s2b)
        g = GS128(Z.reshape(_L * _VP, 128), gidx)
        o = _out_mlp(g, rb(b_s2b), fs, mask, pfs, gfs,
                     W_o0, rb(b_o0), W_o1, rb(b_o1), W_o2, rb(b_o2))
        return o

    outs = [chain(x[b]) for b in range(_B)]
    return jnp.stack(outs, axis=0)[:, :_V, :]


# speedup vs baseline: 10.0685x; 1.0548x over previous
import functools

import jax
import jax.numpy as jnp
from jax import lax
from jax.experimental import pallas as pl
from jax.experimental.pallas import tpu as pltpu
from jax.experimental.pallas import tpu_sc as plsc

_B = 2
_V = 5000
_L = 9
_VP = 5120
_BR = 512
_NRB = _VP // _BR

_NC, _NS = 2, 16
_NW = _NC * _NS
_VPW = _VP // _NW


@functools.lru_cache(maxsize=None)
def _make_gather_sum(C, CH):
    NCH = _VPW // CH
    ncg = C // 16
    mesh = plsc.VectorSubcoreMesh(core_axis_name="c", subcore_axis_name="s")

    @functools.partial(
        pl.kernel,
        mesh=mesh,
        out_type=jax.ShapeDtypeStruct((_VP, C), jnp.float32),
        scratch_types=[
            pltpu.VMEM((_L * _VPW,), jnp.int32),
            pltpu.VMEM((3, _L, CH, C), jnp.float32),
            pltpu.VMEM((CH, C), jnp.float32),
            pltpu.SemaphoreType.DMA,
            pltpu.SemaphoreType.DMA,
            pltpu.SemaphoreType.DMA,
        ],
    )
    def gs(table_hbm, gidx_hbm, out_hbm, idx_v, buf_v, acc_v, sem0, sem1, sem2):
        wid = lax.axis_index("s") * _NC + lax.axis_index("c")
        pltpu.sync_copy(gidx_hbm.at[wid], idx_v)
        sems = (sem0, sem1, sem2)

        def fire(c):
            return [
                pltpu.async_copy(
                    table_hbm.at[idx_v.at[pl.ds(j * _VPW + c * CH, CH)]],
                    buf_v.at[c % 3, j],
                    sems[c % 3],
                )
                for j in range(_L)
            ]

        pends = {c: fire(c) for c in range(min(2, NCH))}
        for c in range(NCH):
            for cp in pends.pop(c):
                cp.wait()
            if c + 2 < NCH:
                pends[c + 2] = fire(c + 2)

            def rbody(r, _, _p=c % 3):
                for cb in range(ncg):
                    co = cb * 16
                    acc = buf_v[_p, 0, r, pl.ds(co, 16)]
                    for j in range(1, _L):
                        acc = acc + buf_v[_p, j, r, pl.ds(co, 16)]
                    acc_v[r, pl.ds(co, 16)] = acc
                return 0

            lax.fori_loop(0, CH, rbody, 0)
            pltpu.sync_copy(acc_v, out_hbm.at[pl.ds(wid * _VPW + c * CH, CH)])

    return gs


def _k1_body(x_ref, wp_ref, w1_ref, m_ref, pfs_ref, z_ref):
    h = jnp.dot(x_ref[...], wp_ref[...], preferred_element_type=jnp.float32)
    h = jnp.maximum(h, 0.0) * m_ref[...]
    pfs_ref[...] = h
    for j in range(_L):
        z_ref[j] = jnp.dot(h, w1_ref[j], preferred_element_type=jnp.float32)


def _k1(xp, wp, w1, mask):
    K = xp.shape[1]
    C = w1.shape[2]
    return pl.pallas_call(
        _k1_body,
        grid=(_NRB,),
        in_specs=[
            pl.BlockSpec((_BR, K), lambda i: (i, 0)),
            pl.BlockSpec((K, 256), lambda i: (0, 0)),
            pl.BlockSpec((_L, 256, C), lambda i: (0, 0, 0)),
            pl.BlockSpec((_BR, 1), lambda i: (i, 0)),
        ],
        out_specs=[
            pl.BlockSpec((_BR, 256), lambda i: (i, 0)),
            pl.BlockSpec((_L, _BR, C), lambda i: (0, i, 0)),
        ],
        out_shape=[
            jax.ShapeDtypeStruct((_VP, 256), jnp.float32),
            jax.ShapeDtypeStruct((_L, _VP, C), jnp.float32),
        ],
    )(xp, wp, w1, mask)


def _act_mm_body(g_ref, b_ref, m_ref, w_ref, z_ref):
    h = jnp.maximum(g_ref[...] + b_ref[...], 0.0) * m_ref[...]
    for j in range(_L):
        z_ref[j] = jnp.dot(h, w_ref[j], preferred_element_type=jnp.float32)


def _act_mm(g, b, mask, w):
    C = g.shape[1]
    CO = w.shape[2]
    return pl.pallas_call(
        _act_mm_body,
        grid=(_NRB,),
        in_specs=[
            pl.BlockSpec((_BR, C), lambda i: (i, 0)),
            pl.BlockSpec((1, C), lambda i: (0, 0)),
            pl.BlockSpec((_BR, 1), lambda i: (i, 0)),
            pl.BlockSpec((_L, C, CO), lambda i: (0, 0, 0)),
        ],
        out_specs=pl.BlockSpec((_L, _BR, CO), lambda i: (0, i, 0)),
        out_shape=jax.ShapeDtypeStruct((_L, _VP, CO), jnp.float32),
    )(g, b, mask, w)


def _res_mm_body(g_ref, b_ref, r_ref, m_ref, w_ref, fs_ref, z_ref):
    h = jnp.maximum(g_ref[...] + b_ref[...] + r_ref[...], 0.0) * m_ref[...]
    fs_ref[...] = h
    for j in range(_L):
        z_ref[j] = jnp.dot(h, w_ref[j], preferred_element_type=jnp.float32)


def _res_mm(g, b, res, mask, w):
    C = g.shape[1]
    CO = w.shape[2]
    return pl.pallas_call(
        _res_mm_body,
        grid=(_NRB,),
        in_specs=[
            pl.BlockSpec((_BR, C), lambda i: (i, 0)),
            pl.BlockSpec((1, C), lambda i: (0, 0)),
            pl.BlockSpec((_BR, C), lambda i: (i, 0)),
            pl.BlockSpec((_BR, 1), lambda i: (i, 0)),
            pl.BlockSpec((_L, C, CO), lambda i: (0, 0, 0)),
        ],
        out_specs=[
            pl.BlockSpec((_BR, C), lambda i: (i, 0)),
            pl.BlockSpec((_L, _BR, CO), lambda i: (0, i, 0)),
        ],
        out_shape=[
            jax.ShapeDtypeStruct((_VP, C), jnp.float32),
            jax.ShapeDtypeStruct((_L, _VP, CO), jnp.float32),
        ],
    )(g, b, res, mask, w)


def _mid_body(g_ref, b_ref, r_ref, m_ref, wmid_ref, wn_ref, fs_ref, z_ref):
    h = jnp.maximum(g_ref[...] + b_ref[...] + r_ref[...], 0.0) * m_ref[...]
    fs = jnp.dot(h, wmid_ref[...], preferred_element_type=jnp.float32)
    fs_ref[...] = fs
    for j in range(_L):
        z_ref[j] = jnp.dot(fs, wn_ref[j], preferred_element_type=jnp.float32)


def _mid(g, b, res, mask, wmid, wn):
    CO = wn.shape[2]
    return pl.pallas_call(
        _mid_body,
        grid=(_NRB,),
        in_specs=[
            pl.BlockSpec((_BR, 256), lambda i: (i, 0)),
            pl.BlockSpec((1, 256), lambda i: (0, 0)),
            pl.BlockSpec((_BR, 256), lambda i: (i, 0)),
            pl.BlockSpec((_BR, 1), lambda i: (i, 0)),
            pl.BlockSpec((256, 128), lambda i: (0, 0)),
            pl.BlockSpec((_L, 128, CO), lambda i: (0, 0, 0)),
        ],
        out_specs=[
            pl.BlockSpec((_BR, 128), lambda i: (i, 0)),
            pl.BlockSpec((_L, _BR, CO), lambda i: (0, i, 0)),
        ],
        out_shape=[
            jax.ShapeDtypeStruct((_VP, 128), jnp.float32),
            jax.ShapeDtypeStruct((_L, _VP, CO), jnp.float32),
        ],
    )(g, b, res, mask, wmid, wn)


def _max_body(fs_ref, m_ref, o_ref):
    o_ref[...] = jnp.max(
        jnp.where(m_ref[...] > 0.0, fs_ref[...], -1e30), axis=0, keepdims=True
    )


def _gmax(fs, mask):
    return pl.pallas_call(
        _max_body,
        in_specs=[
            pl.BlockSpec((_VP, 128), lambda: (0, 0)),
            pl.BlockSpec((_VP, 1), lambda: (0, 0)),
        ],
        out_specs=pl.BlockSpec((1, 128), lambda: (0, 0)),
        out_shape=jax.ShapeDtypeStruct((1, 128), jnp.float32),
    )(fs, mask)


def _out_body(g_ref, b_ref, r_ref, m_ref, pfs_ref, gfs_ref,
              wo0_ref, bo0_ref, wo1_ref, bo1_ref, wo2_ref, bo2_ref, o_ref):
    fs3 = jnp.maximum(g_ref[...] + b_ref[...] + r_ref[...], 0.0) * m_ref[...]
    t = jnp.dot(pfs_ref[...], wo0_ref[0:256], preferred_element_type=jnp.float32)
    t = t + jnp.dot(fs3, wo0_ref[256:384], preferred_element_type=jnp.float32)
    t = t + jnp.dot(gfs_ref[...], wo0_ref[384:512], preferred_element_type=jnp.float32)
    o = jnp.maximum(t + bo0_ref[...], 0.0)
    o = jnp.maximum(
        jnp.dot(o, wo1_ref[...], preferred_element_type=jnp.float32) + bo1_ref[...], 0.0
    )
    o_ref[...] = jnp.dot(o, wo2_ref[...], preferred_element_type=jnp.float32) + bo2_ref[...]


def _out_mlp(g, b, res, mask, pfs, gfs, wo0, bo0, wo1, bo1, wo2, bo2):
    row = lambda i: (i, 0)
    full = lambda i: (0, 0)
    return pl.pallas_call(
        _out_body,
        grid=(_NRB,),
        in_specs=[
            pl.BlockSpec((_BR, 128), row),
            pl.BlockSpec((1, 128), full),
            pl.BlockSpec((_BR, 128), row),
            pl.BlockSpec((_BR, 1), row),
            pl.BlockSpec((_BR, 256), row),
            pl.BlockSpec((1, 128), full),
            pl.BlockSpec((512, 256), full),
            pl.BlockSpec((1, 256), full),
            pl.BlockSpec((256, 128), full),
            pl.BlockSpec((1, 128), full),
            pl.BlockSpec((128, 3), full),
            pl.BlockSpec((1, 3), full),
        ],
        out_specs=pl.BlockSpec((_BR, 3), row),
        out_shape=jax.ShapeDtypeStruct((_VP, 3), jnp.float32),
    )(g, b, res, mask, pfs, gfs, wo0, bo0, wo1, bo1, wo2, bo2)


def _w9(W, C):
    return W.reshape(_L, C, C)


def kernel(x, spiral_idx, W_point, W_r1a, b_r1a, W_r1b, b_r1b, W_mid,
           W_s0a, b_s0a, W_s0b, b_s0b, W_s1a, b_s1a, W_s1b, b_s1b,
           W_s2a, b_s2a, W_s2b, b_s2b, W_o0, b_o0, W_o1, b_o1, W_o2, b_o2):
    f32 = jnp.float32
    wp = jnp.pad(W_point, ((0, 1), (0, 0)))
    mask = (jnp.arange(_VP) < _V).astype(f32)[:, None]

    idx_pad = jnp.pad(spiral_idx, ((0, _VP - _V - 1), (0, 0)))
    offs = jnp.arange(_L, dtype=idx_pad.dtype)[None, :] * _VP
    gidx = idx_pad + offs
    gidx = gidx.reshape(_NW, _VPW, _L).transpose(0, 2, 1).reshape(_NW, _L * _VPW)
    gidx = gidx.astype(jnp.int32)

    rb = lambda v: v.reshape(1, -1)
    GS256 = _make_gather_sum(256, 16)
    GS128 = _make_gather_sum(128, 32)
    w_r1a, w_r1b = _w9(W_r1a, 256), _w9(W_r1b, 256)
    w_s0a, w_s0b = _w9(W_s0a, 128), _w9(W_s0b, 128)
    w_s1a, w_s1b = _w9(W_s1a, 128), _w9(W_s1b, 128)
    w_s2a, w_s2b = _w9(W_s2a, 128), _w9(W_s2b, 128)

    def chain(xb):
        xp = jnp.pad(xb, ((0, _VP - _V), (0, 1)))
        pfs, Z = _k1(xp, wp, w_r1a, mask)
        g = GS256(Z.reshape(_L * _VP, -1), gidx)
        Z = _act_mm(g, rb(b_r1a), mask, w_r1b)
        g = GS256(Z.reshape(_L * _VP, -1), gidx)
        fs, Z = _mid(g, rb(b_r1b), pfs, mask, W_mid, w_s0a)
        gfs = _gmax(fs, mask)
        g = GS128(Z.reshape(_L * _VP, -1), gidx)
        Z = _act_mm(g, rb(b_s0a), mask, w_s0b)
        g = GS128(Z.reshape(_L * _VP, -1), gidx)
        fs, Z = _res_mm(g, rb(b_s0b), fs, mask, w_s1a)
        g = GS128(Z.reshape(_L * _VP, -1), gidx)
        Z = _act_mm(g, rb(b_s1a), mask, w_s1b)
        g = GS128(Z.reshape(_L * _VP, -1), gidx)
        fs, Z = _res_mm(g, rb(b_s1b), fs, mask, w_s2a)
        g = GS128(Z.reshape(_L * _VP, -1), gidx)
        Z = _act_mm(g, rb(b_s2a), mask, w_s2b)
        g = GS128(Z.reshape(_L * _VP, -1), gidx)
        o = _out_mlp(g, rb(b_s2b), fs, mask, pfs, gfs,
                     W_o0, rb(b_o0), W_o1, rb(b_o1), W_o2, rb(b_o2))
        return o

    outs = [chain(x[b]) for b in range(_B)]
    return jnp.stack(outs, axis=0)[:, :_V, :]
